# Initial kernel scaffold; baseline (speedup 1.0000x reference)
import jax
import jax.numpy as jnp
from jax.experimental import pallas as pl


def kernel(data, Wq, bq, Wk, bk, Wv, bv, We, Wskip, bskip, Wc, bc, W1, b1, W2, b2):
    raise NotImplementedError("write your pallas kernel here")


# speedup vs baseline: 8.8814x; 8.8814x over previous
import jax
import jax.numpy as jnp
from jax.experimental import pallas as pl

_G = 48
_N = _G * _G
_H = 4
_C = 32
_HC = _H * _C
_RAD = 0.5
_K = 4
_NACT = 128
_HID = 2 * (_G - 1) * (_G - 1)
_B = 8

_RB = 384
_NBLK = _N // _RB

_JB = 512
_NJ = (_HID + _JB - 1) // _JB


def _attn_body(xrow_ref, xT_ref, x_ref, Wq_ref, bq_ref, Wk_ref, bk_ref,
               Wv_ref, bv_ref, We_ref, Ws_ref, bs_ref, o_ref):
    i = pl.program_id(1)
    xr = xrow_ref[0]
    xr = jnp.where(jnp.isnan(xr), -10.0, xr)
    xT = xT_ref[0]
    xT = jnp.where(jnp.isnan(xT), -10.0, xT)
    x = x_ref[0]
    x = jnp.where(jnp.isnan(x), -10.0, x)

    sqr = jnp.sum(xr * xr, axis=1, keepdims=True)
    sqa = jnp.sum(xT * xT, axis=0, keepdims=True)
    mm = jnp.dot(xr, xT)
    d2 = (sqr + sqa) - 2.0 * mm
    d2 = jnp.maximum(d2, 0.0)
    colid = jax.lax.broadcasted_iota(jnp.int32, (_RB, _N), 1)
    rowid = jax.lax.broadcasted_iota(jnp.int32, (_RB, _N), 0) + i * _RB
    d2 = jnp.where(colid == rowid, d2 + 1e10, d2)

    q = jnp.dot(xr, Wq_ref[...]) + bq_ref[...]
    We = We_ref[...]
    sqc = jnp.sqrt(jnp.float32(_C))

    alphas = []
    vs = []
    valids = []
    for _p in range(_K):
        m = jnp.min(d2, axis=1, keepdims=True)
        eq = d2 == m
        amin = jnp.min(jnp.where(eq, colid, _N), axis=1, keepdims=True)
        onehot = colid == amin
        d2 = jnp.where(onehot, 3e10, d2)
        dist = jnp.sqrt(jnp.maximum(m, 1e-12))
        valid = dist < _RAD
        xj = jax.lax.dot(onehot.astype(jnp.float32), x,
                         precision=jax.lax.Precision.HIGHEST)
        e = dist * We
        kf = jnp.dot(xj, Wk_ref[...]) + bk_ref[...] + e
        vf = jnp.dot(xj, Wv_ref[...]) + bv_ref[...] + e
        qk = q * kf
        ah = []
        for h in range(_H):
            s = jnp.sum(qk[:, h * _C:(h + 1) * _C], axis=1, keepdims=True)
            ah.append(jnp.where(valid, s / sqc, -1e9))
        alphas.append(ah)
        vs.append(vf)
        valids.append(valid.astype(jnp.float32))

    outs = []
    for h in range(_H):
        mx = alphas[0][h]
        for p in range(1, _K):
            mx = jnp.maximum(mx, alphas[p][h])
        es = [jnp.exp(alphas[p][h] - mx) for p in range(_K)]
        den = es[0] + es[1] + es[2] + es[3]
        acc = None
        for p in range(_K):
            w = es[p] / den * valids[p]
            t = w * vs[p][:, h * _C:(h + 1) * _C]
            acc = t if acc is None else acc + t
        outs.append(acc)
    out = jnp.concatenate(outs, axis=1)
    out = out + jnp.dot(xr, Ws_ref[...]) + bs_ref[...]
    o_ref[0] = jnp.maximum(out, 0.0)


def _conv_body(pix_ref, Wcr_ref, bc_ref, o_ref):
    pix = pix_ref[0]
    S = jnp.dot(pix, Wcr_ref[...])
    Sp = jnp.concatenate([S, jnp.zeros((_G + 1, 8), jnp.float32)], axis=0)
    T = (Sp[0:_N, 0:2] + Sp[1:_N + 1, 2:4]
         + Sp[_G:_N + _G, 4:6] + Sp[_G + 1:_N + _G + 1, 6:8])
    o_ref[0] = T + bc_ref[...]


def _mlp_body(z_ref, W1_ref, b1_ref, W2_ref, b2_ref, o_ref):
    j = pl.program_id(0)
    h = jnp.dot(z_ref[...], W1_ref[...]) + b1_ref[...]
    h = jnp.maximum(h, 0.0)
    col = jax.lax.broadcasted_iota(jnp.int32, (1, _JB), 1) + j * _JB
    h = jnp.where(col < _HID, h, 0.0)
    roww = jax.lax.broadcasted_iota(jnp.int32, (_JB, 1), 0) + j * _JB
    w2 = jnp.where(roww < _HID, W2_ref[...], 0.0)
    part = jnp.dot(h, w2)

    @pl.when(j == 0)
    def _():
        o_ref[...] = part

    @pl.when(j > 0)
    def _():
        o_ref[...] = o_ref[...] + part

    @pl.when(j == _NJ - 1)
    def _():
        logits = o_ref[...] + b2_ref[...]
        mxl = jnp.max(logits, axis=1, keepdims=True)
        ex = jnp.exp(logits - mxl)
        o_ref[...] = ex / jnp.sum(ex, axis=1, keepdims=True)


def kernel(data, Wq, bq, Wk, bk, Wv, bv, We, Wskip, bskip, Wc, bc, W1, b1, W2, b2):
    f32 = jnp.float32
    dataT = jnp.swapaxes(data, 1, 2)

    out = pl.pallas_call(
        _attn_body,
        grid=(_B, _NBLK),
        in_specs=[
            pl.BlockSpec((1, _RB, 2), lambda b, i: (b, i, 0)),
            pl.BlockSpec((1, 2, _N), lambda b, i: (b, 0, 0)),
            pl.BlockSpec((1, _N, 2), lambda b, i: (b, 0, 0)),
            pl.BlockSpec((2, _HC), lambda b, i: (0, 0)),
            pl.BlockSpec((1, _HC), lambda b, i: (0, 0)),
            pl.BlockSpec((2, _HC), lambda b, i: (0, 0)),
            pl.BlockSpec((1, _HC), lambda b, i: (0, 0)),
            pl.BlockSpec((2, _HC), lambda b, i: (0, 0)),
            pl.BlockSpec((1, _HC), lambda b, i: (0, 0)),
            pl.BlockSpec((1, _HC), lambda b, i: (0, 0)),
            pl.BlockSpec((2, _HC), lambda b, i: (0, 0)),
            pl.BlockSpec((1, _HC), lambda b, i: (0, 0)),
        ],
        out_specs=pl.BlockSpec((1, _RB, _HC), lambda b, i: (b, i, 0)),
        out_shape=jax.ShapeDtypeStruct((_B, _N, _HC), f32),
    )(data, dataT, data, Wq, bq.reshape(1, _HC), Wk, bk.reshape(1, _HC),
      Wv, bv.reshape(1, _HC), We, Wskip, bskip.reshape(1, _HC))

    pix = out.reshape(_B, _HC, _N).transpose(0, 2, 1)
    Wcr = Wc.transpose(2, 3, 0, 1).reshape(2 * _K, _HC).T

    T = pl.pallas_call(
        _conv_body,
        grid=(_B,),
        in_specs=[
            pl.BlockSpec((1, _N, _HC), lambda b: (b, 0, 0)),
            pl.BlockSpec((_HC, 8), lambda b: (0, 0)),
            pl.BlockSpec((1, 2), lambda b: (0, 0)),
        ],
        out_specs=pl.BlockSpec((1, _N, 2), lambda b: (b, 0, 0)),
        out_shape=jax.ShapeDtypeStruct((_B, _N, 2), f32),
    )(pix, Wcr, bc.reshape(1, 2))

    z = (T.transpose(0, 2, 1).reshape(_B, 2, _G, _G)[:, :, :_G - 1, :_G - 1]
         .reshape(_B, _HID))

    res = pl.pallas_call(
        _mlp_body,
        grid=(_NJ,),
        in_specs=[
            pl.BlockSpec((_B, _HID), lambda j: (0, 0)),
            pl.BlockSpec((_HID, _JB), lambda j: (0, j)),
            pl.BlockSpec((1, _JB), lambda j: (0, j)),
            pl.BlockSpec((_JB, _NACT), lambda j: (j, 0)),
            pl.BlockSpec((1, _NACT), lambda j: (0, 0)),
        ],
        out_specs=pl.BlockSpec((_B, _NACT), lambda j: (0, 0)),
        out_shape=jax.ShapeDtypeStruct((_B, _NACT), f32),
    )(z, W1, b1.reshape(1, _HID), W2, b2.reshape(1, _NACT))
    return res


# speedup vs baseline: 22.6873x; 2.5545x over previous
import jax
import jax.numpy as jnp
from jax.experimental import pallas as pl
from jax.experimental.pallas import tpu as pltpu

_G = 48
_N = _G * _G
_H = 4
_C = 32
_HC = _H * _C
_RAD = 0.5
_K = 4
_NACT = 128
_HID = 2 * (_G - 1) * (_G - 1)
_B = 8

_RB = 384
_NBLK = _N // _RB

_JB = 512
_NJ = (_HID + _JB - 1) // _JB


def _attn_body(xrow_ref, xT_ref, Wq_ref, bq_ref, Wk_ref, bk_ref,
               Wv_ref, bv_ref, We_ref, Ws_ref, bs_ref, o_ref):
    i = pl.program_id(1)
    xr = xrow_ref[0]
    xr = jnp.where(jnp.isnan(xr), -10.0, xr)
    xT = xT_ref[0]
    xT = jnp.where(jnp.isnan(xT), -10.0, xT)

    sqr = jnp.sum(xr * xr, axis=1, keepdims=True)
    sqa = jnp.sum(xT * xT, axis=0, keepdims=True)
    mm = jnp.dot(xr, xT)
    d2 = (sqr + sqa) - 2.0 * mm
    d2 = jnp.maximum(d2, 0.0)
    colid = jax.lax.broadcasted_iota(jnp.int32, (_RB, _N), 1)
    rowid = jax.lax.broadcasted_iota(jnp.int32, (_RB, _N), 0) + i * _RB
    d2 = jnp.where(colid == rowid, d2 + 1e10, d2)

    q = jnp.dot(xr, Wq_ref[...]) + bq_ref[...]
    We = We_ref[...]
    sqc = jnp.sqrt(jnp.float32(_C))

    alphas = []
    vs = []
    valids = []
    for _p in range(_K):
        m = jnp.min(d2, axis=1, keepdims=True)
        eq = d2 == m
        amin = jnp.min(jnp.where(eq, colid, _N), axis=1, keepdims=True)
        onehot = colid == amin
        d2 = jnp.where(onehot, 3e10, d2)
        dist = jnp.sqrt(jnp.maximum(m, 1e-12))
        valid = dist < _RAD
        x0j = jnp.min(jnp.where(onehot, xT[0:1, :], jnp.inf),
                      axis=1, keepdims=True)
        x1j = jnp.min(jnp.where(onehot, xT[1:2, :], jnp.inf),
                      axis=1, keepdims=True)
        xj = jnp.concatenate([x0j, x1j], axis=1)
        e = dist * We
        kf = jnp.dot(xj, Wk_ref[...]) + bk_ref[...] + e
        vf = jnp.dot(xj, Wv_ref[...]) + bv_ref[...] + e
        qk = q * kf
        ah = []
        for h in range(_H):
            s = jnp.sum(qk[:, h * _C:(h + 1) * _C], axis=1, keepdims=True)
            ah.append(jnp.where(valid, s / sqc, -1e9))
        alphas.append(ah)
        vs.append(vf)
        valids.append(valid.astype(jnp.float32))

    outs = []
    for h in range(_H):
        mx = alphas[0][h]
        for p in range(1, _K):
            mx = jnp.maximum(mx, alphas[p][h])
        es = [jnp.exp(alphas[p][h] - mx) for p in range(_K)]
        den = es[0] + es[1] + es[2] + es[3]
        acc = None
        for p in range(_K):
            w = es[p] / den * valids[p]
            t = w * vs[p][:, h * _C:(h + 1) * _C]
            acc = t if acc is None else acc + t
        outs.append(acc)
    out = jnp.concatenate(outs, axis=1)
    out = out + jnp.dot(xr, Ws_ref[...]) + bs_ref[...]
    o_ref[0] = jnp.maximum(out, 0.0)


def _conv_body(pix_ref, Wcr_ref, bc_ref, o_ref):
    pix = pix_ref[0]
    S = jnp.dot(pix, Wcr_ref[...])
    Sp = jnp.concatenate([S, jnp.zeros((_G + 1, 8), jnp.float32)], axis=0)
    T = (Sp[0:_N, 0:2] + Sp[1:_N + 1, 2:4]
         + Sp[_G:_N + _G, 4:6] + Sp[_G + 1:_N + _G + 1, 6:8])
    o_ref[0] = T + bc_ref[...]


def _mlp_body(z_ref, W1_ref, b1_ref, W2_ref, b2_ref, o_ref):
    j = pl.program_id(0)
    h = jnp.dot(z_ref[...], W1_ref[...]) + b1_ref[...]
    h = jnp.maximum(h, 0.0)
    col = jax.lax.broadcasted_iota(jnp.int32, (1, _JB), 1) + j * _JB
    h = jnp.where(col < _HID, h, 0.0)
    roww = jax.lax.broadcasted_iota(jnp.int32, (_JB, 1), 0) + j * _JB
    w2 = jnp.where(roww < _HID, W2_ref[...], 0.0)
    part = jnp.dot(h, w2)

    @pl.when(j == 0)
    def _():
        o_ref[...] = part

    @pl.when(j > 0)
    def _():
        o_ref[...] = o_ref[...] + part

    @pl.when(j == _NJ - 1)
    def _():
        logits = o_ref[...] + b2_ref[...]
        mxl = jnp.max(logits, axis=1, keepdims=True)
        ex = jnp.exp(logits - mxl)
        o_ref[...] = ex / jnp.sum(ex, axis=1, keepdims=True)


def kernel(data, Wq, bq, Wk, bk, Wv, bv, We, Wskip, bskip, Wc, bc, W1, b1, W2, b2):
    f32 = jnp.float32
    dataT = jnp.swapaxes(data, 1, 2)

    out = pl.pallas_call(
        _attn_body,
        grid=(_B, _NBLK),
        in_specs=[
            pl.BlockSpec((1, _RB, 2), lambda b, i: (b, i, 0)),
            pl.BlockSpec((1, 2, _N), lambda b, i: (b, 0, 0)),
            pl.BlockSpec((2, _HC), lambda b, i: (0, 0)),
            pl.BlockSpec((1, _HC), lambda b, i: (0, 0)),
            pl.BlockSpec((2, _HC), lambda b, i: (0, 0)),
            pl.BlockSpec((1, _HC), lambda b, i: (0, 0)),
            pl.BlockSpec((2, _HC), lambda b, i: (0, 0)),
            pl.BlockSpec((1, _HC), lambda b, i: (0, 0)),
            pl.BlockSpec((1, _HC), lambda b, i: (0, 0)),
            pl.BlockSpec((2, _HC), lambda b, i: (0, 0)),
            pl.BlockSpec((1, _HC), lambda b, i: (0, 0)),
        ],
        out_specs=pl.BlockSpec((1, _RB, _HC), lambda b, i: (b, i, 0)),
        out_shape=jax.ShapeDtypeStruct((_B, _N, _HC), f32),
        compiler_params=pltpu.CompilerParams(
            dimension_semantics=("parallel", "parallel")),
    )(data, dataT, Wq, bq.reshape(1, _HC), Wk, bk.reshape(1, _HC),
      Wv, bv.reshape(1, _HC), We, Wskip, bskip.reshape(1, _HC))

    pix = out.reshape(_B, _HC, _N).transpose(0, 2, 1)
    Wcr = Wc.transpose(2, 3, 0, 1).reshape(2 * _K, _HC).T

    T = pl.pallas_call(
        _conv_body,
        grid=(_B,),
        in_specs=[
            pl.BlockSpec((1, _N, _HC), lambda b: (b, 0, 0)),
            pl.BlockSpec((_HC, 8), lambda b: (0, 0)),
            pl.BlockSpec((1, 2), lambda b: (0, 0)),
        ],
        out_specs=pl.BlockSpec((1, _N, 2), lambda b: (b, 0, 0)),
        out_shape=jax.ShapeDtypeStruct((_B, _N, 2), f32),
        compiler_params=pltpu.CompilerParams(
            dimension_semantics=("parallel",)),
    )(pix, Wcr, bc.reshape(1, 2))

    z = (T.transpose(0, 2, 1).reshape(_B, 2, _G, _G)[:, :, :_G - 1, :_G - 1]
         .reshape(_B, _HID))

    res = pl.pallas_call(
        _mlp_body,
        grid=(_NJ,),
        in_specs=[
            pl.BlockSpec((_B, _HID), lambda j: (0, 0)),
            pl.BlockSpec((_HID, _JB), lambda j: (0, j)),
            pl.BlockSpec((1, _JB), lambda j: (0, j)),
            pl.BlockSpec((_JB, _NACT), lambda j: (j, 0)),
            pl.BlockSpec((1, _NACT), lambda j: (0, 0)),
        ],
        out_specs=pl.BlockSpec((_B, _NACT), lambda j: (0, 0)),
        out_shape=jax.ShapeDtypeStruct((_B, _NACT), f32),
    )(z, W1, b1.reshape(1, _HID), W2, b2.reshape(1, _NACT))
    return res
